# Initial kernel scaffold; baseline (speedup 1.0000x reference)
import functools
import jax
import jax.numpy as jnp
from jax import lax
from jax.experimental import pallas as pl
from jax.experimental.pallas import tpu as pltpu
from jax.experimental.pallas import tpu_sc as plsc

NC = 2
NS = 16
LANES = 16
NW = NC * NS

EMBED_DIM = 128
DBLK = EMBED_DIM // LANES


def _make_sc_lookup(n_total: int, vocab: int):
    per_w = n_total // NW
    chunk = 256
    n_chunks = per_w // chunk
    assert per_w % chunk == 0 and n_total % NW == 0

    mesh = plsc.VectorSubcoreMesh(
        core_axis_name="c", subcore_axis_name="s",
        num_cores=NC, num_subcores=NS)

    @functools.partial(
        pl.kernel,
        out_type=jax.ShapeDtypeStruct((n_total, EMBED_DIM), jnp.float32),
        mesh=mesh,
        scratch_types=[
            pltpu.VMEM((per_w,), jnp.float32),
            pltpu.VMEM((per_w,), jnp.int32),
            pltpu.VMEM((per_w,), jnp.int32),
            pltpu.VMEM((per_w,), jnp.float32),
            pltpu.VMEM((chunk, EMBED_DIM), jnp.float32),
            pltpu.VMEM((chunk, EMBED_DIM), jnp.float32),
            pltpu.SemaphoreType.DMA,
        ],
    )
    def lookup(x_hbm, w_hbm, out_hbm, x_v, idx1_v, idx2_v, frac_v,
               e1_v, e2_v, sem):
        wid = lax.axis_index("s") * NC + lax.axis_index("c")
        base = wid * per_w
        pltpu.sync_copy(x_hbm.at[pl.ds(base, per_w)], x_v)

        def ix_body(k, _):
            xv = x_v[pl.ds(k * LANES, LANES)]
            i1 = xv.astype(jnp.int32)
            fr = xv - i1.astype(jnp.float32)
            i2 = jnp.minimum(i1 + 1, vocab - 2)
            idx1_v[pl.ds(k * LANES, LANES)] = i1
            idx2_v[pl.ds(k * LANES, LANES)] = i2
            frac_v[pl.ds(k * LANES, LANES)] = fr
            return 0

        lax.fori_loop(0, per_w // LANES, ix_body, 0)

        def chunk_body(c, _):
            cp1 = pltpu.async_copy(
                w_hbm.at[idx1_v.at[pl.ds(c * chunk, chunk)]], e1_v, sem)
            cp2 = pltpu.async_copy(
                w_hbm.at[idx2_v.at[pl.ds(c * chunk, chunk)]], e2_v, sem)
            cp1.wait()
            cp2.wait()

            def j_body(j, _):
                f = frac_v[c * chunk + j]
                for d in range(DBLK):
                    sl = pl.ds(d * LANES, LANES)
                    e1 = e1_v[j, sl]
                    e2 = e2_v[j, sl]
                    e1_v[j, sl] = e1 + (e2 - e1) * f
                return 0

            lax.fori_loop(0, chunk, j_body, 0)
            pltpu.sync_copy(e1_v, out_hbm.at[pl.ds(base + c * chunk, chunk)])
            return 0

        lax.fori_loop(0, n_chunks, chunk_body, 0)

    return lookup


def kernel(x, weight):
    batch, fields = x.shape
    n_total = batch * fields
    vocab = weight.shape[0]
    out = _make_sc_lookup(n_total, vocab)(x.reshape(n_total), weight)
    return out.reshape(batch, fields, EMBED_DIM)


# speedup vs baseline: 3.2009x; 3.2009x over previous
import functools
import jax
import jax.numpy as jnp
from jax import lax
from jax.experimental import pallas as pl
from jax.experimental.pallas import tpu as pltpu
from jax.experimental.pallas import tpu_sc as plsc

NC = 2
NS = 16
LANES = 16
NW = NC * NS

EMBED_DIM = 128
DBLK = EMBED_DIM // LANES


def _make_sc_lookup(n_total: int, vocab: int):
    per_w = n_total // NW
    chunk = 256
    n_chunks = per_w // chunk
    assert per_w % chunk == 0 and n_total % NW == 0

    mesh = plsc.VectorSubcoreMesh(
        core_axis_name="c", subcore_axis_name="s",
        num_cores=NC, num_subcores=NS)

    @functools.partial(
        pl.kernel,
        out_type=jax.ShapeDtypeStruct((n_total, EMBED_DIM), jnp.float32),
        mesh=mesh,
        scratch_types=[
            pltpu.VMEM((per_w,), jnp.float32),
            pltpu.VMEM((per_w,), jnp.int32),
            pltpu.VMEM((per_w,), jnp.int32),
            pltpu.VMEM((per_w,), jnp.float32),
            pltpu.VMEM((chunk, EMBED_DIM), jnp.float32),
            pltpu.VMEM((chunk, EMBED_DIM), jnp.float32),
            pltpu.SemaphoreType.DMA,
        ],
    )
    def lookup(x_hbm, w_hbm, out_hbm, x_v, idx1_v, idx2_v, frac_v,
               e1_v, e2_v, sem):
        wid = lax.axis_index("s") * NC + lax.axis_index("c")
        base = wid * per_w
        pltpu.sync_copy(x_hbm.at[pl.ds(base, per_w)], x_v)

        def ix_body(k, _):
            xv = x_v[pl.ds(k * LANES, LANES)]
            i1 = xv.astype(jnp.int32)
            fr = xv - i1.astype(jnp.float32)
            i2 = jnp.minimum(i1 + 1, vocab - 2)
            idx1_v[pl.ds(k * LANES, LANES)] = i1
            idx2_v[pl.ds(k * LANES, LANES)] = i2
            frac_v[pl.ds(k * LANES, LANES)] = fr
            return 0

        lax.fori_loop(0, per_w // LANES, ix_body, 0)

        def chunk_body(c, _):
            cp1 = pltpu.async_copy(
                w_hbm.at[idx1_v.at[pl.ds(c * chunk, chunk)]], e1_v, sem)
            cp2 = pltpu.async_copy(
                w_hbm.at[idx2_v.at[pl.ds(c * chunk, chunk)]], e2_v, sem)
            cp1.wait()
            cp2.wait()

            def g_body(g, _):
                fv = frac_v[pl.ds(c * chunk + g * LANES, LANES)]
                for lane in range(LANES):
                    j = g * LANES + lane
                    f = fv[lane]
                    for d in range(DBLK):
                        sl = pl.ds(d * LANES, LANES)
                        e1 = e1_v[j, sl]
                        e2 = e2_v[j, sl]
                        e1_v[j, sl] = e1 + (e2 - e1) * f
                return 0

            lax.fori_loop(0, chunk // LANES, g_body, 0)
            pltpu.sync_copy(e1_v, out_hbm.at[pl.ds(base + c * chunk, chunk)])
            return 0

        lax.fori_loop(0, n_chunks, chunk_body, 0)

    return lookup


def kernel(x, weight):
    batch, fields = x.shape
    n_total = batch * fields
    vocab = weight.shape[0]
    out = _make_sc_lookup(n_total, vocab)(x.reshape(n_total), weight)
    return out.reshape(batch, fields, EMBED_DIM)
